# Initial kernel scaffold; baseline (speedup 1.0000x reference)
import jax
import jax.numpy as jnp
from jax.experimental import pallas as pl


def kernel(cls_ids, vec):
    raise NotImplementedError("write your pallas kernel here")


# speedup vs baseline: 2.3320x; 2.3320x over previous
import functools

import jax
import jax.numpy as jnp
from jax import lax
from jax.experimental import pallas as pl
from jax.experimental.pallas import tpu as pltpu
from jax.experimental.pallas import tpu_sc as plsc

_V = 8192
_D = 256
_B = 16384
_NC = 2
_NS = 16
_NW = _NC * _NS
_BPW = _B // _NW
_CHUNK = 128
_NCHUNK = _BPW // _CHUNK

_mesh = plsc.VectorSubcoreMesh(core_axis_name="c", subcore_axis_name="s")


@functools.partial(
    pl.kernel,
    mesh=_mesh,
    out_type=jax.ShapeDtypeStruct((_B, _D), jnp.float32),
    scratch_types=[
        pltpu.VMEM((_NCHUNK, _CHUNK), jnp.int32),
        pltpu.VMEM((_CHUNK, _D), jnp.float32),
        pltpu.VMEM((_CHUNK, _D), jnp.float32),
        pltpu.SemaphoreType.DMA,
        pltpu.SemaphoreType.DMA,
    ],
)
def _sc_gather(idx_hbm, table_hbm, out_hbm, idx_v, buf0, buf1, sem0, sem1):
    wid = lax.axis_index("s") * _NC + lax.axis_index("c")
    base = wid * _BPW
    pltpu.sync_copy(idx_hbm.at[wid], idx_v)
    bufs = (buf0, buf1)
    sems = (sem0, sem1)
    copies = []
    for c in range(_NCHUNK):
        copies.append(
            pltpu.async_copy(table_hbm.at[idx_v.at[c]], bufs[c % 2], sems[c % 2])
        )
        if c >= 1:
            copies[c - 1].wait()
            pltpu.sync_copy(
                bufs[(c - 1) % 2],
                out_hbm.at[pl.ds(base + (c - 1) * _CHUNK, _CHUNK)],
            )
    copies[-1].wait()
    pltpu.sync_copy(
        bufs[(_NCHUNK - 1) % 2],
        out_hbm.at[pl.ds(base + (_NCHUNK - 1) * _CHUNK, _CHUNK)],
    )


def kernel(cls_ids, vec):
    idx3 = cls_ids.reshape(_NW, _NCHUNK, _CHUNK)
    return _sc_gather(idx3, vec)


# speedup vs baseline: 2.3571x; 1.0108x over previous
import functools

import jax
import jax.numpy as jnp
from jax import lax
from jax.experimental import pallas as pl
from jax.experimental.pallas import tpu as pltpu
from jax.experimental.pallas import tpu_sc as plsc

_V = 8192
_D = 256
_B = 16384
_NC = 2
_NS = 16
_NW = _NC * _NS
_BPW = _B // _NW
_CHUNK = 128
_NCHUNK = _BPW // _CHUNK
_NBUF = 3

_mesh = plsc.VectorSubcoreMesh(core_axis_name="c", subcore_axis_name="s")


@functools.partial(
    pl.kernel,
    mesh=_mesh,
    out_type=jax.ShapeDtypeStruct((_B, _D), jnp.float32),
    scratch_types=[
        pltpu.VMEM((_NCHUNK, _CHUNK), jnp.int32),
    ]
    + [pltpu.VMEM((_CHUNK, _D), jnp.float32) for _ in range(_NBUF)]
    + [pltpu.SemaphoreType.DMA for _ in range(2 * _NBUF)],
)
def _sc_gather(idx_hbm, table_hbm, out_hbm, idx_v, *scratch):
    bufs = scratch[:_NBUF]
    gsems = scratch[_NBUF:2 * _NBUF]
    wsems = scratch[2 * _NBUF:]
    wid = lax.axis_index("s") * _NC + lax.axis_index("c")
    base = wid * _BPW
    pltpu.sync_copy(idx_hbm.at[wid], idx_v)
    gcp = [None] * _NCHUNK
    wcp = [None] * _NCHUNK
    for c in range(_NCHUNK):
        b = c % _NBUF
        if c >= _NBUF:
            wcp[c - _NBUF].wait()
        gcp[c] = pltpu.async_copy(table_hbm.at[idx_v.at[c]], bufs[b], gsems[b])
        if c >= 1:
            p = c - 1
            gcp[p].wait()
            wcp[p] = pltpu.async_copy(
                bufs[p % _NBUF],
                out_hbm.at[pl.ds(base + p * _CHUNK, _CHUNK)],
                wsems[p % _NBUF],
            )
    last = _NCHUNK - 1
    gcp[last].wait()
    wcp[last] = pltpu.async_copy(
        bufs[last % _NBUF],
        out_hbm.at[pl.ds(base + last * _CHUNK, _CHUNK)],
        wsems[last % _NBUF],
    )
    for c in range(max(0, _NCHUNK - _NBUF), _NCHUNK):
        wcp[c].wait()


def kernel(cls_ids, vec):
    idx3 = cls_ids.reshape(_NW, _NCHUNK, _CHUNK)
    return _sc_gather(idx3, vec)
